# baseline (speedup 1.0000x reference)
import functools

import jax
import jax.numpy as jnp
from jax import lax
from jax.experimental import pallas as pl
from jax.experimental.pallas import tpu as pltpu
from jax.experimental.pallas import tpu_sc as plsc

NUM_CORES = 2
NUM_SUBCORES = 16
NUM_WORKERS = NUM_CORES * NUM_SUBCORES
BATCH = 16384
EMB = 64
B_PER_W = BATCH // NUM_WORKERS


def _sc_gather_kernel(uids_hbm, aids_hbm, uemb_hbm, aemb_hbm, ub_hbm, ab_hbm,
                      urows_out, arows_out, ub_out, ab_out,
                      uidx_v, aidx_v, urows_v, arows_v, ub_v, ab_v,
                      sem0, sem1, sem2, sem3):
    wid = lax.axis_index("s") * NUM_CORES + lax.axis_index("c")
    base = wid * B_PER_W
    pltpu.sync_copy(uids_hbm.at[pl.ds(base, B_PER_W)], uidx_v)
    pltpu.sync_copy(aids_hbm.at[pl.ds(base, B_PER_W)], aidx_v)
    cp0 = pltpu.async_copy(uemb_hbm.at[uidx_v], urows_v, sem0)
    cp1 = pltpu.async_copy(aemb_hbm.at[aidx_v], arows_v, sem1)
    cp2 = pltpu.async_copy(ub_hbm.at[uidx_v], ub_v, sem2)
    cp3 = pltpu.async_copy(ab_hbm.at[aidx_v], ab_v, sem3)
    cp0.wait()
    cp1.wait()
    cp2.wait()
    cp3.wait()
    pltpu.sync_copy(urows_v, urows_out.at[pl.ds(base, B_PER_W)])
    pltpu.sync_copy(arows_v, arows_out.at[pl.ds(base, B_PER_W)])
    pltpu.sync_copy(ub_v, ub_out.at[pl.ds(base, B_PER_W)])
    pltpu.sync_copy(ab_v, ab_out.at[pl.ds(base, B_PER_W)])


def _tc_finish_kernel(u_ref, a_ref, ub_ref, ab_ref, o_ref):
    dot = jnp.sum(u_ref[...] * a_ref[...], axis=1, keepdims=True)
    x = dot + ub_ref[...] + ab_ref[...]
    o_ref[...] = jax.nn.sigmoid(x)


def kernel(user_ids, anime_ids, user_emb, anime_emb, user_bias, anime_bias):
    mesh = plsc.VectorSubcoreMesh(core_axis_name="c", subcore_axis_name="s")
    gather = pl.kernel(
        _sc_gather_kernel,
        out_type=(
            jax.ShapeDtypeStruct((BATCH, EMB), jnp.float32),
            jax.ShapeDtypeStruct((BATCH, EMB), jnp.float32),
            jax.ShapeDtypeStruct((BATCH,), jnp.float32),
            jax.ShapeDtypeStruct((BATCH,), jnp.float32),
        ),
        mesh=mesh,
        compiler_params=pltpu.CompilerParams(use_tc_tiling_on_sc=False),
        scratch_types=[
            pltpu.VMEM((B_PER_W,), jnp.int32),
            pltpu.VMEM((B_PER_W,), jnp.int32),
            pltpu.VMEM((B_PER_W, EMB), jnp.float32),
            pltpu.VMEM((B_PER_W, EMB), jnp.float32),
            pltpu.VMEM((B_PER_W,), jnp.float32),
            pltpu.VMEM((B_PER_W,), jnp.float32),
            pltpu.SemaphoreType.DMA,
            pltpu.SemaphoreType.DMA,
            pltpu.SemaphoreType.DMA,
            pltpu.SemaphoreType.DMA,
        ],
    )
    urows, arows, ubg, abg = gather(
        user_ids.astype(jnp.int32),
        anime_ids.astype(jnp.int32),
        user_emb,
        anime_emb,
        user_bias.reshape(-1),
        anime_bias.reshape(-1),
    )

    blk = 2048
    out = pl.pallas_call(
        _tc_finish_kernel,
        out_shape=jax.ShapeDtypeStruct((BATCH, 1), jnp.float32),
        grid=(BATCH // blk,),
        in_specs=[
            pl.BlockSpec((blk, EMB), lambda i: (i, 0)),
            pl.BlockSpec((blk, EMB), lambda i: (i, 0)),
            pl.BlockSpec((blk, 1), lambda i: (i, 0)),
            pl.BlockSpec((blk, 1), lambda i: (i, 0)),
        ],
        out_specs=pl.BlockSpec((blk, 1), lambda i: (i, 0)),
    )(urows, arows, ubg.reshape(BATCH, 1), abg.reshape(BATCH, 1))
    return out


# speedup vs baseline: 1.0467x; 1.0467x over previous
import jax
import jax.numpy as jnp
from jax import lax
from jax.experimental import pallas as pl
from jax.experimental.pallas import tpu as pltpu
from jax.experimental.pallas import tpu_sc as plsc

NUM_CORES = 2
NUM_SUBCORES = 16
NUM_WORKERS = NUM_CORES * NUM_SUBCORES
BATCH = 16384
EMB = 64
B_PER_W = BATCH // NUM_WORKERS
HALF_B = B_PER_W // 2


def _bias_kernel(uids, aids, ubflat, abflat, ub_out, ab_out,
                 uid_v, aid_v, ub_v, ab_v, sem, semb):
    wid = lax.axis_index("s") * NUM_CORES + lax.axis_index("c")
    base = wid * B_PER_W
    pltpu.async_copy(uids.at[pl.ds(base, B_PER_W)], uid_v, sem).wait()
    pltpu.async_copy(aids.at[pl.ds(base, B_PER_W)], aid_v, sem).wait()
    cu = pltpu.async_copy(ubflat.at[uid_v], ub_v, semb)
    ca = pltpu.async_copy(abflat.at[aid_v], ab_v, semb)
    cu.wait()
    ca.wait()
    pltpu.async_copy(ub_v, ub_out.at[pl.ds(base, B_PER_W)], sem).wait()
    pltpu.async_copy(ab_v, ab_out.at[pl.ds(base, B_PER_W)], sem).wait()


def _main_kernel(upair, apair, uids2, aids2, ub2, ab2, out_hbm,
                 uid_v, aid_v, idxu_v, idxa_v, u128_v, a128_v,
                 res_v, ub_v, ab_v, sem, sem_u, sem_a):
    wid = lax.axis_index("s") * NUM_CORES + lax.axis_index("c")

    pltpu.async_copy(uids2.at[wid], uid_v, sem).wait()
    pltpu.async_copy(aids2.at[wid], aid_v, sem).wait()
    cb_u = pltpu.async_copy(ub2.at[wid], ub_v, sem)
    cb_a = pltpu.async_copy(ab2.at[wid], ab_v, sem)

    @pl.loop(0, B_PER_W, step=16)
    def _(k):
        sl = pl.ds(k, 16)
        idxu_v[sl] = lax.shift_right_logical(uid_v[sl], 1)
        idxa_v[sl] = lax.shift_right_logical(aid_v[sl], 1)

    lane = lax.iota(jnp.int32, 16)

    for h in range(2):
        hbase = h * HALF_B
        cu = pltpu.async_copy(
            upair.at[idxu_v.at[pl.ds(hbase, HALF_B)]], u128_v, sem_u)
        ca = pltpu.async_copy(
            apair.at[idxa_v.at[pl.ds(hbase, HALF_B)]], a128_v, sem_a)
        cu.wait()
        ca.wait()

        @pl.loop(0, HALF_B, step=16)
        def _(k):
            vu = uid_v[pl.ds(hbase + k, 16)]
            va = aid_v[pl.ds(hbase + k, 16)]
            out_reg = jnp.zeros((16,), jnp.float32)
            for i in range(16):
                uoff = (vu[i] & 1) * 64
                aoff = (va[i] & 1) * 64
                acc = (u128_v[k + i, pl.ds(uoff, 16)] *
                       a128_v[k + i, pl.ds(aoff, 16)])
                for t in range(1, 4):
                    acc = acc + (u128_v[k + i, pl.ds(uoff + t * 16, 16)] *
                                 a128_v[k + i, pl.ds(aoff + t * 16, 16)])
                s = lax.reduce_sum(acc, axes=(0,))
                out_reg = jnp.where(lane == i, s, out_reg)
            res_v[pl.ds(hbase + k, 16)] = out_reg

    cb_u.wait()
    cb_a.wait()

    @pl.loop(0, B_PER_W, step=16)
    def _(k):
        sl = pl.ds(k, 16)
        x = res_v[sl] + ub_v[sl] + ab_v[sl]
        res_v[sl] = 1.0 / (1.0 + jnp.exp(-x))

    pltpu.async_copy(res_v, out_hbm.at[wid], sem).wait()


def kernel(user_ids, anime_ids, user_emb, anime_emb, user_bias, anime_bias):
    n_user = user_emb.shape[0]
    n_anime = anime_emb.shape[0]
    uids32 = user_ids.astype(jnp.int32)
    aids32 = anime_ids.astype(jnp.int32)
    upair = user_emb.reshape(n_user // 2, 2 * EMB)
    apair = anime_emb.reshape(n_anime // 2, 2 * EMB)
    ubflat = user_bias.reshape(-1)
    abflat = anime_bias.reshape(-1)

    mesh = plsc.VectorSubcoreMesh(core_axis_name="c", subcore_axis_name="s")

    bias_gather = pl.kernel(
        _bias_kernel,
        out_type=(
            jax.ShapeDtypeStruct((BATCH,), jnp.float32),
            jax.ShapeDtypeStruct((BATCH,), jnp.float32),
        ),
        mesh=mesh,
        scratch_types=[
            pltpu.VMEM((B_PER_W,), jnp.int32),
            pltpu.VMEM((B_PER_W,), jnp.int32),
            pltpu.VMEM((B_PER_W,), jnp.float32),
            pltpu.VMEM((B_PER_W,), jnp.float32),
            pltpu.SemaphoreType.DMA,
            pltpu.SemaphoreType.DMA,
        ],
    )
    ubg, abg = bias_gather(uids32, aids32, ubflat, abflat)

    main = pl.kernel(
        _main_kernel,
        out_type=jax.ShapeDtypeStruct((NUM_WORKERS, B_PER_W), jnp.float32),
        mesh=mesh,
        compiler_params=pltpu.CompilerParams(
            use_tc_tiling_on_sc=True, needs_layout_passes=False),
        scratch_types=[
            pltpu.VMEM((B_PER_W,), jnp.int32),
            pltpu.VMEM((B_PER_W,), jnp.int32),
            pltpu.VMEM((B_PER_W,), jnp.int32),
            pltpu.VMEM((B_PER_W,), jnp.int32),
            pltpu.VMEM((HALF_B, 2 * EMB), jnp.float32),
            pltpu.VMEM((HALF_B, 2 * EMB), jnp.float32),
            pltpu.VMEM((B_PER_W,), jnp.float32),
            pltpu.VMEM((B_PER_W,), jnp.float32),
            pltpu.VMEM((B_PER_W,), jnp.float32),
            pltpu.SemaphoreType.DMA,
            pltpu.SemaphoreType.DMA,
            pltpu.SemaphoreType.DMA,
        ],
    )
    out = main(
        upair, apair,
        uids32.reshape(NUM_WORKERS, B_PER_W),
        aids32.reshape(NUM_WORKERS, B_PER_W),
        ubg.reshape(NUM_WORKERS, B_PER_W),
        abg.reshape(NUM_WORKERS, B_PER_W),
    )
    return out.reshape(BATCH, 1)
